# baseline (speedup 1.0000x reference)
import functools

import jax
import jax.numpy as jnp
from jax import lax
from jax.experimental import pallas as pl
from jax.experimental.pallas import tpu as pltpu
from jax.experimental.pallas import tpu_sc as plsc

B = 4096
C = 26
EMB = 64
NUM_NUM = 13
NUM_TEXT = 4
TEXT_IN = 768

NC = 2
NS = 16
NW = NC * NS

TOTAL = B * C
PER_W = TOTAL // NW


def _sc_gather(idx, tidx, table3):
    mesh = plsc.VectorSubcoreMesh(core_axis_name="c", subcore_axis_name="s")

    NB = 3
    CH = 64
    NCH = PER_W // CH
    GRP = CH // 16
    QE = EMB // 16

    @functools.partial(
        pl.kernel,
        mesh=mesh,
        compiler_params=pltpu.CompilerParams(
            use_tc_tiling_on_sc=False, needs_layout_passes=False
        ),
        out_type=jax.ShapeDtypeStruct((TOTAL, EMB), jnp.float32),
        scratch_types=[
            pltpu.VMEM((PER_W,), jnp.int32),
            pltpu.VMEM((PER_W,), jnp.int32),
            pltpu.VMEM((NB, CH, 8, EMB), jnp.float32),
            pltpu.VMEM((2, CH, EMB), jnp.float32),
            pltpu.SemaphoreType.DMA((NB,)),
            pltpu.SemaphoreType.DMA((2,)),
            pltpu.SemaphoreType.DMA,
        ],
    )
    def k(idx_hbm, tidx_hbm, table_hbm, out_hbm, idx_v, tidx_v, slabs_v,
          out_v, ssem, osem, isem):
        wid = lax.axis_index("s") * NC + lax.axis_index("c")
        base_row = wid * PER_W
        pltpu.async_copy(
            idx_hbm.at[pl.ds(base_row, PER_W)], idx_v, isem
        ).wait()
        pltpu.async_copy(
            tidx_hbm.at[pl.ds(base_row, PER_W)], tidx_v, isem
        ).wait()

        def slab_stream(j, p):
            return pltpu.make_async_copy(
                table_hbm.at[tidx_v.at[pl.ds(j * CH, CH)]],
                slabs_v.at[p],
                ssem.at[p],
            )

        def outcopy(j, ob):
            return pltpu.make_async_copy(
                out_v.at[ob],
                out_hbm.at[pl.ds(base_row + j * CH, CH)],
                osem.at[ob],
            )

        for j in range(NB - 1):
            slab_stream(j, j).start()

        def body(j, carry):
            p = lax.rem(j, NB)
            ob = lax.rem(j, 2)

            @pl.when(j + NB - 1 < NCH)
            def _():
                slab_stream(j + NB - 1, lax.rem(j + NB - 1, NB)).start()

            @pl.when(j >= 2)
            def _():
                outcopy(j - 2, ob).wait()

            slab_stream(j, p).wait()
            slab = slabs_v.at[p]
            dst = out_v.at[ob]
            for g in range(GRP):
                idx16 = idx_v[pl.ds(j * CH + g * 16, 16)]
                m16 = jnp.bitwise_and(idx16, 7)
                r16 = lax.iota(jnp.int32, 16) + g * 16
                for col in range(EMB):
                    c16 = jnp.full((16,), col, jnp.int32)
                    vals = plsc.load_gather(slab, [r16, m16, c16])
                    plsc.store_scatter(dst, [r16, c16], vals)
            outcopy(j, ob).start()
            return carry

        lax.fori_loop(0, NCH, body, 0)
        for t in range(2):
            j = NCH - 2 + t
            outcopy(j, lax.rem(jnp.int32(j), 2)).wait()

    return k(idx, tidx, table3)


def _tc_dense_body(num_ref, text_ref, dir_ref, anc_ref, w_ref, out_ref):
    num = num_ref[...]
    out_ref[:, :NUM_NUM, :] = (
        num[:, :, None] * dir_ref[...][None] + anc_ref[...][None]
    )
    t = text_ref[...]
    for i in range(NUM_TEXT):
        out_ref[:, NUM_NUM + i, :] = jnp.dot(
            t[:, i, :], w_ref[i], preferred_element_type=jnp.float32
        )


def _tc_dense(numerical_inputs, text_inputs, direction, anchor, text_w):
    BB = 512
    grid = (B // BB,)
    return pl.pallas_call(
        _tc_dense_body,
        grid=grid,
        in_specs=[
            pl.BlockSpec((BB, NUM_NUM), lambda b: (b, 0)),
            pl.BlockSpec((BB, NUM_TEXT, TEXT_IN), lambda b: (b, 0, 0)),
            pl.BlockSpec((NUM_NUM, EMB), lambda b: (0, 0)),
            pl.BlockSpec((NUM_NUM, EMB), lambda b: (0, 0)),
            pl.BlockSpec((NUM_TEXT, TEXT_IN, EMB), lambda b: (0, 0, 0)),
        ],
        out_specs=pl.BlockSpec((BB, NUM_NUM + NUM_TEXT, EMB), lambda b: (b, 0, 0)),
        out_shape=jax.ShapeDtypeStruct((B, NUM_NUM + NUM_TEXT, EMB), jnp.float32),
    )(numerical_inputs, text_inputs, direction, anchor, text_w)


def kernel(categorical_inputs, numerical_inputs, text_inputs, table,
           numerical_direction, numerical_anchor, text_w, offsets):
    idx = (categorical_inputs + offsets).reshape(TOTAL)
    tidx = idx >> 3
    table3 = table.reshape(table.shape[0] // 8, 8, EMB)
    cat = _sc_gather(idx, tidx, table3).reshape(B, C, EMB)
    dense = _tc_dense(numerical_inputs, text_inputs, numerical_direction,
                      numerical_anchor, text_w)
    return jnp.concatenate((cat, dense), axis=1)


# speedup vs baseline: 1.7842x; 1.7842x over previous
import functools

import jax
import jax.numpy as jnp
from jax import lax
from jax.experimental import pallas as pl
from jax.experimental.pallas import tpu as pltpu
from jax.experimental.pallas import tpu_sc as plsc

B = 4096
C = 26
EMB = 64
NUM_NUM = 13
NUM_TEXT = 4
TEXT_IN = 768

NC = 2
NS = 16
NW = NC * NS

TOTAL = B * C
PER_W = TOTAL // NW


def _sc_gather(idx, table3):
    mesh = plsc.VectorSubcoreMesh(core_axis_name="c", subcore_axis_name="s")

    G = 16
    SCH = 256
    NGR = SCH // G
    NSC = PER_W // SCH
    QE = EMB // 16

    @functools.partial(
        pl.kernel,
        mesh=mesh,
        compiler_params=pltpu.CompilerParams(needs_layout_passes=False),
        out_type=jax.ShapeDtypeStruct((TOTAL, EMB), jnp.float32),
        scratch_types=[
            pltpu.VMEM((SCH,), jnp.int32),
            pltpu.VMEM((G, 8, EMB), jnp.float32),
            pltpu.VMEM((2, SCH, EMB), jnp.float32),
            pltpu.SemaphoreType.DMA((G,)),
            pltpu.SemaphoreType.DMA((2,)),
            pltpu.SemaphoreType.DMA,
        ],
    )
    def k(idx_hbm, table_hbm, out_hbm, idx_v, tiles_v, out_v, tsem, osem,
          isem):
        wid = lax.axis_index("s") * NC + lax.axis_index("c")
        base_row = wid * PER_W
        LANES = lax.iota(jnp.int32, 16)

        def slab_copy(t, p):
            return pltpu.make_async_copy(
                table_hbm.at[t], tiles_v.at[p], tsem.at[p]
            )

        def fire_group(g):
            idx16 = idx_v[pl.ds(g * G, G)]
            ms = []
            for l in range(G):
                s = jnp.max(jnp.where(LANES == l, idx16, 0))
                slab_copy(s >> 3, l).start()
                ms.append(jnp.bitwise_and(s, 7))
            return tuple(ms)

        def extract_group(gprev, ms, ob):
            for l in range(G):
                slab_copy(jnp.int32(0), l).wait()
                r = gprev * G + l
                for c in range(QE):
                    out_v[ob, r, pl.ds(c * 16, 16)] = (
                        tiles_v[l, ms[l], pl.ds(c * 16, 16)]
                    )

        def outcopy(sc, ob):
            return pltpu.make_async_copy(
                out_v.at[ob],
                out_hbm.at[pl.ds(base_row + sc * SCH, SCH)],
                osem.at[ob],
            )

        def super_chunk(sc, carry):
            ob = lax.rem(sc, 2)
            pltpu.async_copy(
                idx_hbm.at[pl.ds(base_row + sc * SCH, SCH)], idx_v, isem
            ).wait()

            @pl.when(sc >= 2)
            def _():
                outcopy(sc - 2, ob).wait()

            ms0 = fire_group(jnp.int32(0))

            def body(g, ms):
                idx16 = idx_v[pl.ds(g * G, G)]
                new_ms = []
                for l in range(G):
                    slab_copy(jnp.int32(0), l).wait()
                    r = (g - 1) * G + l
                    for c in range(QE):
                        out_v[ob, r, pl.ds(c * 16, 16)] = (
                            tiles_v[l, ms[l], pl.ds(c * 16, 16)]
                        )
                    s = jnp.max(jnp.where(LANES == l, idx16, 0))
                    slab_copy(s >> 3, l).start()
                    new_ms.append(jnp.bitwise_and(s, 7))
                return tuple(new_ms)

            ms_last = lax.fori_loop(1, NGR, body, ms0)
            extract_group(jnp.int32(NGR - 1), ms_last, ob)
            outcopy(sc, ob).start()
            return carry

        lax.fori_loop(0, NSC, super_chunk, 0)
        for t in range(2):
            outcopy(NSC - 2 + t, lax.rem(jnp.int32(NSC - 2 + t), 2)).wait()

    return k(idx, table3)


def _tc_dense_body(num_ref, text_ref, dir_ref, anc_ref, w_ref, out_ref):
    num = num_ref[...]
    out_ref[:, :NUM_NUM, :] = (
        num[:, :, None] * dir_ref[...][None] + anc_ref[...][None]
    )
    t = text_ref[...]
    for i in range(NUM_TEXT):
        out_ref[:, NUM_NUM + i, :] = jnp.dot(
            t[:, i, :], w_ref[i], preferred_element_type=jnp.float32
        )


def _tc_dense(numerical_inputs, text_inputs, direction, anchor, text_w):
    BB = 512
    grid = (B // BB,)
    return pl.pallas_call(
        _tc_dense_body,
        grid=grid,
        in_specs=[
            pl.BlockSpec((BB, NUM_NUM), lambda b: (b, 0)),
            pl.BlockSpec((BB, NUM_TEXT, TEXT_IN), lambda b: (b, 0, 0)),
            pl.BlockSpec((NUM_NUM, EMB), lambda b: (0, 0)),
            pl.BlockSpec((NUM_NUM, EMB), lambda b: (0, 0)),
            pl.BlockSpec((NUM_TEXT, TEXT_IN, EMB), lambda b: (0, 0, 0)),
        ],
        out_specs=pl.BlockSpec((BB, NUM_NUM + NUM_TEXT, EMB), lambda b: (b, 0, 0)),
        out_shape=jax.ShapeDtypeStruct((B, NUM_NUM + NUM_TEXT, EMB), jnp.float32),
    )(numerical_inputs, text_inputs, direction, anchor, text_w)


def kernel(categorical_inputs, numerical_inputs, text_inputs, table,
           numerical_direction, numerical_anchor, text_w, offsets):
    idx = (categorical_inputs + offsets).reshape(TOTAL)
    table3 = table.reshape(table.shape[0] // 8, 8, EMB)
    cat = _sc_gather(idx, table3).reshape(B, C, EMB)
    dense = _tc_dense(numerical_inputs, text_inputs, numerical_direction,
                      numerical_anchor, text_w)
    return jnp.concatenate((cat, dense), axis=1)
